# baseline (speedup 1.0000x reference)
import functools

import jax
import jax.numpy as jnp
from jax import lax
from jax.experimental import pallas as pl
from jax.experimental.pallas import tpu as pltpu
from jax.experimental.pallas import tpu_sc as plsc

N_NODES = 10000
N_PAD = 10240
N_EDGES = 320000
D = 128
NPH = 5
NH = N_PAD // NPH
ACC_R = 2176
TRASH = NH

NC = 2
NS = 16
NT = NC * NS
CH = 128
E_PAD = 327680
EPT = E_PAD // NT
NCH = EPT // CH
RPT = N_PAD // NS
ZR = 128
N_BIG = 16384

_mesh = plsc.VectorSubcoreMesh(core_axis_name="c", subcore_axis_name="s")


@functools.partial(
    pl.kernel,
    mesh=_mesh,
    out_type=(
        jax.ShapeDtypeStruct((NC, N_PAD, D), jnp.float32),
        jax.ShapeDtypeStruct((NT * N_PAD,), jnp.float32),
    ),
    scratch_types=[
        pltpu.VMEM((NCH, CH), jnp.int32),
        pltpu.VMEM((NCH, CH), jnp.int32),
        pltpu.VMEM((NCH, CH), jnp.float32),
        pltpu.VMEM((CH, 16), jnp.float32),
        pltpu.VMEM((CH, D), jnp.float32),
        pltpu.VMEM((CH, D), jnp.float32),
        pltpu.VMEM((CH,), jnp.int32),
        pltpu.VMEM((ACC_R // NS, D), jnp.float32),
        pltpu.VMEM((N_PAD,), jnp.float32),
        pltpu.VMEM_SHARED((ACC_R, D), jnp.float32),
        pltpu.SemaphoreType.DMA,
    ],
)
def _sc_pass(table_hbm, src_hbm, dst_hbm, w_hbm, wrep_hbm, agg_hbm, deg_hbm,
             srcv, dstv, wv, wbuf, xr, outr, idxb, zb, degl, accsh, sem):
    cid = lax.axis_index("c")
    sid = lax.axis_index("s")
    tid = cid * NS + sid
    zero16 = jnp.zeros((16,), jnp.float32)
    zrows = ACC_R // NS
    drows = NH // NS

    pltpu.sync_copy(src_hbm.at[tid], srcv)
    pltpu.sync_copy(dst_hbm.at[tid], dstv)
    pltpu.sync_copy(w_hbm.at[tid], wv)

    def _zrow(i, _):
        for j in range(D // 16):
            zb[i, pl.ds(j * 16, 16)] = zero16
        return 0

    lax.fori_loop(0, zrows, _zrow, 0)

    def _zdeg(i, _):
        degl[pl.ds(i * 16, 16)] = zero16
        return 0

    lax.fori_loop(0, N_PAD // 16, _zdeg, 0)
    iota16 = lax.iota(jnp.int32, 16)

    for p in range(NPH):
        pltpu.sync_copy(zb, accsh.at[pl.ds(sid * zrows, zrows)])
        plsc.subcore_barrier()

        def _chunk(i, _):
            pltpu.sync_copy(wrep_hbm.at[pl.ds((tid * NCH + i) * CH, CH)],
                            wbuf)
            pltpu.async_copy(table_hbm.at[srcv.at[i]], xr, sem).wait()
            for j in range(CH // 16):
                dj = dstv[i, pl.ds(j * 16, 16)] - p * NH
                ok = (dj >= 0) & (dj < NH)
                idxb[pl.ds(j * 16, 16)] = jnp.where(ok, dj, TRASH)
            for r in range(CH):
                wr = wbuf[r, pl.ds(0, 16)]
                for j in range(D // 16):
                    outr[r, pl.ds(j * 16, 16)] = (
                        xr[r, pl.ds(j * 16, 16)] * wr)
            pltpu.sync_copy(outr, accsh.at[idxb], add=True)
            if p == 0:
                for g in range(CH // 16):
                    dvec = dstv[i, pl.ds(g * 16, 16)]
                    wvec = wv[i, pl.ds(g * 16, 16)]
                    for l in range(16):
                        dsc = dvec[l]
                        base = dsc & -16
                        lane = dsc & 15
                        plsc.addupdate(
                            degl.at[pl.ds(base, 16)],
                            jnp.where(iota16 == lane, wvec[l], 0.0))
            return 0

        lax.fori_loop(0, NCH, _chunk, 0)
        plsc.subcore_barrier()

        pltpu.sync_copy(
            accsh.at[pl.ds(sid * drows, drows)],
            agg_hbm.at[cid, pl.ds(p * NH + sid * drows, drows)])
        if p < NPH - 1:
            plsc.subcore_barrier()

    pltpu.sync_copy(degl, deg_hbm.at[pl.ds(tid * N_PAD, N_PAD)])


def _wbody(ea_ref, ew_ref, eb_ref, o_ref):
    z = jnp.dot(ea_ref[...], ew_ref[...], preferred_element_type=jnp.float32)
    o_ref[...] = jax.nn.softplus(z + eb_ref[...])


_BE = 8000


def _edge_w(edge_attr, ew, eb):
    return pl.pallas_call(
        _wbody,
        grid=(N_EDGES // _BE,),
        in_specs=[
            pl.BlockSpec((_BE, 16), lambda i: (i, 0)),
            pl.BlockSpec((16, 2), lambda i: (0, 0)),
            pl.BlockSpec((1, 2), lambda i: (0, 0)),
        ],
        out_specs=pl.BlockSpec((_BE, 2), lambda i: (i, 0)),
        out_shape=jax.ShapeDtypeStruct((N_EDGES, 2), jnp.float32),
    )(edge_attr, ew, eb)


def _dsbody(dp_ref, o_ref):
    o_ref[...] = jnp.sum(dp_ref[...], axis=0, keepdims=True)


def _degsum(dp):
    return pl.pallas_call(
        _dsbody,
        grid=(N_PAD // 2048,),
        in_specs=[pl.BlockSpec((NT, 2048), lambda i: (0, i))],
        out_specs=pl.BlockSpec((1, 2048), lambda i: (0, i)),
        out_shape=jax.ShapeDtypeStruct((1, N_PAD), jnp.float32),
    )(dp)


def _fbody(ap_ref, dp_ref, t_ref, wt_ref, b_ref, o_ref, *, relu):
    agg = ap_ref[0] + ap_ref[1] - dp_ref[...] * t_ref[...]
    y = jnp.dot(agg, wt_ref[...], preferred_element_type=jnp.float32)
    y = y + b_ref[...]
    o_ref[...] = jnp.maximum(y, 0.0) if relu else y


_BN = 1000


def _finish(ap, dp, table, wt, b, relu):
    return pl.pallas_call(
        functools.partial(_fbody, relu=relu),
        grid=(N_NODES // _BN,),
        in_specs=[
            pl.BlockSpec((NC, _BN, D), lambda i: (0, i, 0)),
            pl.BlockSpec((_BN, 1), lambda i: (i, 0)),
            pl.BlockSpec((_BN, D), lambda i: (i, 0)),
            pl.BlockSpec((D, D), lambda i: (0, 0)),
            pl.BlockSpec((1, D), lambda i: (0, 0)),
        ],
        out_specs=pl.BlockSpec((_BN, D), lambda i: (i, 0)),
        out_shape=jax.ShapeDtypeStruct((N_NODES, D), jnp.float32),
    )(ap, dp, table, wt, b)


def kernel(x, edge_index, edge_attr, lin1_W, lin1_b, e1_W, e1_b,
           lin2_W, lin2_b, e2_W, e2_b):
    src = edge_index[0].astype(jnp.int32)
    dst = edge_index[1].astype(jnp.int32)

    ew = jnp.concatenate([e1_W, e2_W], axis=0).T
    eb = jnp.concatenate([e1_b, e2_b], axis=0)[None, :]
    w12 = _edge_w(edge_attr, ew, eb)

    pad = E_PAD - N_EDGES
    wp = jnp.concatenate([w12, jnp.zeros((pad, 2), jnp.float32)], axis=0)
    w1 = wp[:, 0].reshape(NT, NCH, CH)
    w2 = wp[:, 1].reshape(NT, NCH, CH)
    w1r = jnp.broadcast_to(wp[:, 0:1], (E_PAD, 16))
    w2r = jnp.broadcast_to(wp[:, 1:2], (E_PAD, 16))
    zpad = jnp.zeros((pad,), jnp.int32)
    srcp = jnp.concatenate([src, zpad]).reshape(NT, NCH, CH)
    dstp = jnp.concatenate([dst, zpad]).reshape(NT, NCH, CH)

    ap1, dg1 = _sc_pass(x, srcp, dstp, w1, w1r)
    dg1c = _degsum(dg1.reshape(NT, N_PAD)).reshape(N_PAD, 1)[:N_NODES]
    h = _finish(ap1[:, :N_NODES], dg1c, x, lin1_W.T, lin1_b[None, :],
                relu=True)
    h_pad = jnp.concatenate([h, jnp.zeros((N_BIG - N_NODES, D), jnp.float32)])
    ap2, dg2 = _sc_pass(h_pad, srcp, dstp, w2, w2r)
    dg2c = _degsum(dg2.reshape(NT, N_PAD)).reshape(N_PAD, 1)[:N_NODES]
    out = _finish(ap2[:, :N_NODES], dg2c, h, lin2_W.T, lin2_b[None, :],
                  relu=False)
    return out


# speedup vs baseline: 1.3394x; 1.3394x over previous
import functools

import jax
import jax.numpy as jnp
from jax import lax
from jax.experimental import pallas as pl
from jax.experimental.pallas import tpu as pltpu
from jax.experimental.pallas import tpu_sc as plsc

N_NODES = 10000
N_PAD = 10240
N_EDGES = 320000
D = 128
NPH = 5
NH = N_PAD // NPH
ACC_R = 2176
TRASH = NH

NC = 2
NS = 16
NT = NC * NS
CH = 128
E_PAD = 327680
EPT = E_PAD // NT
NCH = EPT // CH
RPT = N_PAD // NS
ZR = 128
N_BIG = 16384

_mesh = plsc.VectorSubcoreMesh(core_axis_name="c", subcore_axis_name="s")


@functools.partial(
    pl.kernel,
    mesh=_mesh,
    out_type=(
        jax.ShapeDtypeStruct((NC, N_PAD, D), jnp.float32),
        jax.ShapeDtypeStruct((NT * N_PAD,), jnp.float32),
    ),
    scratch_types=[
        pltpu.VMEM((NCH, CH), jnp.int32),
        pltpu.VMEM((NCH, CH), jnp.int32),
        pltpu.VMEM((NCH, CH), jnp.float32),
        pltpu.VMEM((2, CH, D), jnp.float32),
        pltpu.VMEM((2, CH, D), jnp.float32),
        pltpu.VMEM((2, CH), jnp.int32),
        pltpu.VMEM((N_PAD,), jnp.float32),
        pltpu.VMEM_SHARED((ACC_R, D), jnp.float32),
        pltpu.SemaphoreType.DMA,
        pltpu.SemaphoreType.DMA,
        pltpu.SemaphoreType.DMA,
        pltpu.SemaphoreType.DMA,
    ],
)
def _sc_pass(table_hbm, src_hbm, dst_hbm, w_hbm, agg_hbm, deg_hbm,
             srcv, dstv, wv, xr, outr, idxb, degl, accsh,
             sg0, sg1, ss0, ss1):
    cid = lax.axis_index("c")
    sid = lax.axis_index("s")
    tid = cid * NS + sid
    zero16 = jnp.zeros((16,), jnp.float32)
    zrows = ACC_R // NS
    drows = NH // NS
    sem_g = (sg0, sg1)
    sem_s = (ss0, ss1)

    pltpu.sync_copy(src_hbm.at[tid], srcv)
    pltpu.sync_copy(dst_hbm.at[tid], dstv)
    pltpu.sync_copy(w_hbm.at[tid], wv)

    def _zdeg(i, _):
        degl[pl.ds(i * 16, 16)] = zero16
        return 0

    lax.fori_loop(0, N_PAD // 16, _zdeg, 0)
    iota16 = lax.iota(jnp.int32, 16)

    def _gather(i, b):
        pltpu.async_copy(table_hbm.at[srcv.at[i]], xr.at[b], sem_g[b])

    def _gather_wait(i, b):
        pltpu.make_async_copy(table_hbm.at[srcv.at[i]], xr.at[b],
                              sem_g[b]).wait()

    def _scatter_start(b):
        pltpu.async_copy(outr.at[b], accsh.at[idxb.at[b]], sem_s[b],
                         add=True)

    def _scatter_wait(b):
        pltpu.make_async_copy(outr.at[b], accsh.at[idxb.at[b]],
                              sem_s[b]).wait()

    def _compute(i, b, p):
        for j in range(CH // 16):
            dj = dstv[i, pl.ds(j * 16, 16)] - p * NH
            ok = (dj >= 0) & (dj < NH)
            idxb[b, pl.ds(j * 16, 16)] = jnp.where(ok, dj, TRASH)

        def _scale16(g, _):
            w16 = wv[i, pl.ds(g * 16, 16)]
            for r16 in range(16):
                r = g * 16 + r16
                wr = jnp.take(w16, jnp.full((16,), r16, jnp.int32))
                for j in range(D // 16):
                    outr[b, r, pl.ds(j * 16, 16)] = (
                        xr[b, r, pl.ds(j * 16, 16)] * wr)
            return 0

        lax.fori_loop(0, CH // 16, _scale16, 0)

        @pl.when(p == 0)
        def _():
            def _deg16(g, _):
                dvec = dstv[i, pl.ds(g * 16, 16)]
                wvec = wv[i, pl.ds(g * 16, 16)]
                for l in range(16):
                    dsc = dvec[l]
                    base = dsc & -16
                    lane = dsc & 15
                    plsc.addupdate(
                        degl.at[pl.ds(base, 16)],
                        jnp.where(iota16 == lane, wvec[l], 0.0))
                return 0

            lax.fori_loop(0, CH // 16, _deg16, 0)

    def _phase(p, _):
        def _zrow(i, _):
            for j in range(D // 16):
                outr[0, i, pl.ds(j * 16, 16)] = zero16
            return 0

        lax.fori_loop(0, CH, _zrow, 0)
        pltpu.sync_copy(outr.at[0], accsh.at[pl.ds(sid * zrows, CH)])
        pltpu.sync_copy(outr.at[0, pl.ds(0, zrows - CH)],
                        accsh.at[pl.ds(sid * zrows + CH, zrows - CH)])
        plsc.subcore_barrier()

        _gather(0, 0)

        def _pair(k, _):
            for b in range(2):
                i = 2 * k + b

                @pl.when(k > 0)
                def _():
                    _scatter_wait(b)

                @pl.when(i + 1 < NCH)
                def _():
                    _gather(i + 1, 1 - b)

                _gather_wait(i, b)
                _compute(i, b, p)
                _scatter_start(b)
            return 0

        lax.fori_loop(0, NCH // 2, _pair, 0)
        _scatter_wait(0)
        _scatter_wait(1)
        plsc.subcore_barrier()

        pltpu.sync_copy(
            accsh.at[pl.ds(sid * drows, drows)],
            agg_hbm.at[cid, pl.ds(p * NH + sid * drows, drows)])
        plsc.subcore_barrier()
        return 0

    lax.fori_loop(0, NPH, _phase, 0)

    pltpu.sync_copy(degl, deg_hbm.at[pl.ds(tid * N_PAD, N_PAD)])


def _wbody(ea_ref, ew_ref, eb_ref, o_ref):
    z = jnp.dot(ea_ref[...], ew_ref[...], preferred_element_type=jnp.float32)
    o_ref[...] = jax.nn.softplus(z + eb_ref[...])


_BE = 8000


def _edge_w(edge_attr, ew, eb):
    return pl.pallas_call(
        _wbody,
        grid=(N_EDGES // _BE,),
        in_specs=[
            pl.BlockSpec((_BE, 16), lambda i: (i, 0)),
            pl.BlockSpec((16, 2), lambda i: (0, 0)),
            pl.BlockSpec((1, 2), lambda i: (0, 0)),
        ],
        out_specs=pl.BlockSpec((_BE, 2), lambda i: (i, 0)),
        out_shape=jax.ShapeDtypeStruct((N_EDGES, 2), jnp.float32),
    )(edge_attr, ew, eb)


def _dsbody(dp_ref, o_ref):
    o_ref[...] = jnp.sum(dp_ref[...], axis=0, keepdims=True)


def _degsum(dp):
    return pl.pallas_call(
        _dsbody,
        grid=(N_PAD // 2048,),
        in_specs=[pl.BlockSpec((NT, 2048), lambda i: (0, i))],
        out_specs=pl.BlockSpec((1, 2048), lambda i: (0, i)),
        out_shape=jax.ShapeDtypeStruct((1, N_PAD), jnp.float32),
    )(dp)


def _fbody(ap_ref, dp_ref, t_ref, wt_ref, b_ref, o_ref, *, relu):
    agg = ap_ref[0] + ap_ref[1] - dp_ref[...] * t_ref[...]
    y = jnp.dot(agg, wt_ref[...], preferred_element_type=jnp.float32)
    y = y + b_ref[...]
    o_ref[...] = jnp.maximum(y, 0.0) if relu else y


_BN = 1000


def _finish(ap, dp, table, wt, b, relu):
    return pl.pallas_call(
        functools.partial(_fbody, relu=relu),
        grid=(N_NODES // _BN,),
        in_specs=[
            pl.BlockSpec((NC, _BN, D), lambda i: (0, i, 0)),
            pl.BlockSpec((_BN, 1), lambda i: (i, 0)),
            pl.BlockSpec((_BN, D), lambda i: (i, 0)),
            pl.BlockSpec((D, D), lambda i: (0, 0)),
            pl.BlockSpec((1, D), lambda i: (0, 0)),
        ],
        out_specs=pl.BlockSpec((_BN, D), lambda i: (i, 0)),
        out_shape=jax.ShapeDtypeStruct((N_NODES, D), jnp.float32),
    )(ap, dp, table, wt, b)


def kernel(x, edge_index, edge_attr, lin1_W, lin1_b, e1_W, e1_b,
           lin2_W, lin2_b, e2_W, e2_b):
    src = edge_index[0].astype(jnp.int32)
    dst = edge_index[1].astype(jnp.int32)

    ew = jnp.concatenate([e1_W, e2_W], axis=0).T
    eb = jnp.concatenate([e1_b, e2_b], axis=0)[None, :]
    w12 = _edge_w(edge_attr, ew, eb)

    pad = E_PAD - N_EDGES
    wp = jnp.concatenate([w12, jnp.zeros((pad, 2), jnp.float32)], axis=0)
    w1 = wp[:, 0].reshape(NT, NCH, CH)
    w2 = wp[:, 1].reshape(NT, NCH, CH)
    zpad = jnp.zeros((pad,), jnp.int32)
    srcp = jnp.concatenate([src, zpad]).reshape(NT, NCH, CH)
    dstp = jnp.concatenate([dst, zpad]).reshape(NT, NCH, CH)

    ap1, dg1 = _sc_pass(x, srcp, dstp, w1)
    dg1c = _degsum(dg1.reshape(NT, N_PAD)).reshape(N_PAD, 1)[:N_NODES]
    h = _finish(ap1[:, :N_NODES], dg1c, x, lin1_W.T, lin1_b[None, :],
                relu=True)
    h_pad = jnp.concatenate([h, jnp.zeros((N_BIG - N_NODES, D), jnp.float32)])
    ap2, dg2 = _sc_pass(h_pad, srcp, dstp, w2)
    dg2c = _degsum(dg2.reshape(NT, N_PAD)).reshape(N_PAD, 1)[:N_NODES]
    out = _finish(ap2[:, :N_NODES], dg2c, h, lin2_W.T, lin2_b[None, :],
                  relu=False)
    return out


# speedup vs baseline: 2.8678x; 2.1410x over previous
import functools

import jax
import jax.numpy as jnp
from jax import lax
from jax.experimental import pallas as pl
from jax.experimental.pallas import tpu as pltpu
from jax.experimental.pallas import tpu_sc as plsc

N_NODES = 10000
N_PAD = 10240
N_EDGES = 320000
D = 128
NPH = 2
NH = 6144
ACC_R = 6272
TRASH = NH

NC = 2
NS = 16
NT = NC * NS
CH = 64
E_PAD = 327680
EPT = E_PAD // NT
NCH = EPT // CH
RPT = N_PAD // NS
ZR = 128
N_BIG = 16384

_mesh = plsc.VectorSubcoreMesh(core_axis_name="c", subcore_axis_name="s")


@functools.partial(
    pl.kernel,
    mesh=_mesh,
    out_type=(
        jax.ShapeDtypeStruct((NC, N_BIG, D), jnp.float32),
        jax.ShapeDtypeStruct((NT * N_PAD,), jnp.float32),
    ),
    scratch_types=[
        pltpu.VMEM((NCH, CH), jnp.int32),
        pltpu.VMEM((2, CH), jnp.int32),
        pltpu.VMEM((NCH, CH), jnp.float32),
        pltpu.VMEM((2, CH, D), jnp.float32),
        pltpu.VMEM((1, CH, D), jnp.float32),
        pltpu.VMEM((1, CH), jnp.int32),
        pltpu.VMEM((N_PAD,), jnp.float32),
        pltpu.VMEM_SHARED((ACC_R, D), jnp.float32),
        pltpu.SemaphoreType.DMA,
        pltpu.SemaphoreType.DMA,
        pltpu.SemaphoreType.DMA,
        pltpu.SemaphoreType.DMA,
    ],
)
def _sc_pass(table_hbm, sd_hbm, w_hbm, agg_hbm, deg_hbm,
             sdv, sck, wv, xr, outr, idxb, degl, accsh,
             sg0, sg1, ss0, ss1):
    cid = lax.axis_index("c")
    sid = lax.axis_index("s")
    tid = cid * NS + sid
    zero16 = jnp.zeros((16,), jnp.float32)
    zrows = ACC_R // NS
    drows = NH // NS
    sem_g = (sg0, sg1)
    sem_s = (ss0, ss1)

    pltpu.sync_copy(sd_hbm.at[tid], sdv)
    pltpu.sync_copy(w_hbm.at[tid], wv)

    def _zdeg(i, _):
        degl[pl.ds(i * 16, 16)] = zero16
        return 0

    lax.fori_loop(0, N_PAD // 16, _zdeg, 0)
    iota16 = lax.iota(jnp.int32, 16)

    def _gather(i, b):
        for j in range(CH // 16):
            sck[b, pl.ds(j * 16, 16)] = (
                sdv[i, pl.ds(j * 16, 16)] & 16383)
        pltpu.async_copy(table_hbm.at[sck.at[b]], xr.at[b], sem_g[b])

    def _gather_wait(i, b):
        pltpu.make_async_copy(table_hbm.at[sck.at[b]], xr.at[b],
                              sem_g[b]).wait()

    def _scatter_start():
        pltpu.async_copy(outr.at[0], accsh.at[idxb.at[0]], sem_s[0],
                         add=True)

    def _scatter_wait():
        pltpu.make_async_copy(outr.at[0], accsh.at[idxb.at[0]],
                              sem_s[0]).wait()

    def _compute(i, b, p):
        for j in range(CH // 16):
            d = sdv[i, pl.ds(j * 16, 16)] >> 14
            dj = d - p * NH
            ok = (dj >= 0) & (dj < NH)
            idxb[0, pl.ds(j * 16, 16)] = jnp.where(ok, dj, TRASH + (d & 127))

        def _scale16(g, _):
            w16 = wv[i, pl.ds(g * 16, 16)]
            for r16 in range(16):
                r = g * 16 + r16
                wr = jnp.take(w16, jnp.full((16,), r16, jnp.int32))
                for j in range(D // 16):
                    outr[0, r, pl.ds(j * 16, 16)] = (
                        xr[b, r, pl.ds(j * 16, 16)] * wr)
            return 0

        lax.fori_loop(0, CH // 16, _scale16, 0)

        @pl.when(p == 0)
        def _():
            def _deg16(g, _):
                dvec = sdv[i, pl.ds(g * 16, 16)] >> 14
                wvec = wv[i, pl.ds(g * 16, 16)]
                for l in range(16):
                    dsc = dvec[l]
                    base = dsc & -16
                    lane = dsc & 15
                    plsc.addupdate(
                        degl.at[pl.ds(base, 16)],
                        jnp.where(iota16 == lane, wvec[l], 0.0))
                return 0

            lax.fori_loop(0, CH // 16, _deg16, 0)

    def _phase(p, _):
        def _zrow(i, _):
            for j in range(D // 16):
                outr[0, i, pl.ds(j * 16, 16)] = zero16
            return 0

        lax.fori_loop(0, CH, _zrow, 0)
        for zo in range(0, zrows, CH):
            zn = min(CH, zrows - zo)
            pltpu.sync_copy(outr.at[0, pl.ds(0, zn)],
                            accsh.at[pl.ds(sid * zrows + zo, zn)])
        plsc.subcore_barrier()

        _gather(0, 0)

        def _pair(k, _):
            for b in range(2):
                i = 2 * k + b

                @pl.when(i > 0)
                def _():
                    _scatter_wait()

                @pl.when(i + 1 < NCH)
                def _():
                    _gather(i + 1, 1 - b)

                _gather_wait(i, b)
                _compute(i, b, p)
                _scatter_start()
            return 0

        lax.fori_loop(0, NCH // 2, _pair, 0)
        _scatter_wait()
        plsc.subcore_barrier()

        pltpu.sync_copy(
            accsh.at[pl.ds(sid * drows, drows)],
            agg_hbm.at[cid, pl.ds(p * NH + sid * drows, drows)])
        plsc.subcore_barrier()
        return 0

    lax.fori_loop(0, NPH, _phase, 0)

    pltpu.sync_copy(degl, deg_hbm.at[pl.ds(tid * N_PAD, N_PAD)])


def _wbody(ea_ref, ew_ref, eb_ref, o_ref):
    z = jnp.dot(ea_ref[...], ew_ref[...], preferred_element_type=jnp.float32)
    o_ref[...] = jax.nn.softplus(z + eb_ref[...])


_BE = 8000


def _edge_w(edge_attr, ew, eb):
    return pl.pallas_call(
        _wbody,
        grid=(N_EDGES // _BE,),
        in_specs=[
            pl.BlockSpec((_BE, 16), lambda i: (i, 0)),
            pl.BlockSpec((16, 2), lambda i: (0, 0)),
            pl.BlockSpec((1, 2), lambda i: (0, 0)),
        ],
        out_specs=pl.BlockSpec((_BE, 2), lambda i: (i, 0)),
        out_shape=jax.ShapeDtypeStruct((N_EDGES, 2), jnp.float32),
    )(edge_attr, ew, eb)


def _dsbody(dp_ref, o_ref):
    o_ref[...] = jnp.sum(dp_ref[...], axis=0, keepdims=True)


def _degsum(dp):
    return pl.pallas_call(
        _dsbody,
        grid=(N_PAD // 2048,),
        in_specs=[pl.BlockSpec((NT, 2048), lambda i: (0, i))],
        out_specs=pl.BlockSpec((1, 2048), lambda i: (0, i)),
        out_shape=jax.ShapeDtypeStruct((1, N_PAD), jnp.float32),
    )(dp)


def _fbody(ap_ref, dp_ref, t_ref, wt_ref, b_ref, o_ref, *, relu):
    agg = ap_ref[0] + ap_ref[1] - dp_ref[...] * t_ref[...]
    y = jnp.dot(agg, wt_ref[...], preferred_element_type=jnp.float32)
    y = y + b_ref[...]
    o_ref[...] = jnp.maximum(y, 0.0) if relu else y


_BN = 1000


def _finish(ap, dp, table, wt, b, relu):
    return pl.pallas_call(
        functools.partial(_fbody, relu=relu),
        grid=(N_NODES // _BN,),
        in_specs=[
            pl.BlockSpec((NC, _BN, D), lambda i: (0, i, 0)),
            pl.BlockSpec((_BN, 1), lambda i: (i, 0)),
            pl.BlockSpec((_BN, D), lambda i: (i, 0)),
            pl.BlockSpec((D, D), lambda i: (0, 0)),
            pl.BlockSpec((1, D), lambda i: (0, 0)),
        ],
        out_specs=pl.BlockSpec((_BN, D), lambda i: (i, 0)),
        out_shape=jax.ShapeDtypeStruct((N_NODES, D), jnp.float32),
    )(ap, dp, table, wt, b)


def kernel(x, edge_index, edge_attr, lin1_W, lin1_b, e1_W, e1_b,
           lin2_W, lin2_b, e2_W, e2_b):
    src = edge_index[0].astype(jnp.int32)
    dst = edge_index[1].astype(jnp.int32)

    ew = jnp.concatenate([e1_W, e2_W], axis=0).T
    eb = jnp.concatenate([e1_b, e2_b], axis=0)[None, :]
    w12 = _edge_w(edge_attr, ew, eb)

    x_pad = jnp.concatenate([x, jnp.zeros((N_BIG - N_NODES, D), jnp.float32)])
    pad = E_PAD - N_EDGES
    wp = jnp.concatenate([w12, jnp.zeros((pad, 2), jnp.float32)], axis=0)
    w1 = wp[:, 0].reshape(NT, NCH, CH)
    w2 = wp[:, 1].reshape(NT, NCH, CH)
    zpad = jnp.zeros((pad,), jnp.int32)
    srcp = jnp.concatenate([src, zpad])
    dstp = jnp.concatenate([dst, zpad])
    sdp = (srcp | (dstp << 14)).reshape(NT, NCH, CH)

    ap1, dg1 = _sc_pass(x_pad, sdp, w1)
    dg1c = _degsum(dg1.reshape(NT, N_PAD)).reshape(N_PAD, 1)[:N_NODES]
    h = _finish(ap1[:, :N_NODES], dg1c, x, lin1_W.T, lin1_b[None, :],
                relu=True)
    h_pad = jnp.concatenate([h, jnp.zeros((N_BIG - N_NODES, D), jnp.float32)])
    ap2, dg2 = _sc_pass(h_pad, sdp, w2)
    dg2c = _degsum(dg2.reshape(NT, N_PAD)).reshape(N_PAD, 1)[:N_NODES]
    out = _finish(ap2[:, :N_NODES], dg2c, h, lin2_W.T, lin2_b[None, :],
                  relu=False)
    return out
